# baseline (speedup 1.0000x reference)
import functools

import jax
import jax.numpy as jnp
from jax import lax
from jax.experimental import pallas as pl
from jax.experimental.pallas import tpu as pltpu
from jax.experimental.pallas import tpu_sc as plsc

NUM_CORES = 2
NUM_SUBCORES = 16
NUM_WORKERS = NUM_CORES * NUM_SUBCORES
LANES = 16
CHUNK = 8
NBUF = 3


def _embed_body(nbatch, seqlen, dim, seqp_hbm, tok_hbm, pos_hbm, out_hbm,
                idx_v, pos_v, tok_v, *sems):
  gs = sems[:NBUF]
  ws = sems[NBUF:]
  pw = seqlen // NUM_WORKERS
  nch = pw // CHUNK
  nvec = dim // LANES

  wid = lax.axis_index("s") * NUM_CORES + lax.axis_index("c")
  p0 = pl.multiple_of(wid * pw, pw)

  pltpu.sync_copy(seqp_hbm.at[wid], idx_v)

  def issue_gathers(k, c):
    return [
        pltpu.async_copy(pos_hbm.at[pl.ds(p0 + c * CHUNK, CHUNK)],
                         pos_v.at[k], gs[k]),
        pltpu.async_copy(tok_hbm.at[idx_v.at[c]], tok_v.at[k], gs[k]),
    ]

  def issue_writes(k, c):
    return [pltpu.async_copy(tok_v.at[k, pl.ds(b * CHUNK, CHUNK)],
                             out_hbm.at[b, pl.ds(p0 + c * CHUNK, CHUNK)],
                             ws[k])
            for b in range(nbatch)]

  UNROLL = 16

  def add_pos(k):
    def row_body(r, carry):
      def vec_body(vv, carry2):
        for u in range(UNROLL):
          sl = pl.ds((vv * UNROLL + u) * LANES, LANES)
          p = pos_v[k, r, sl]
          for b in range(nbatch):
            tok_v[k, b * CHUNK + r, sl] = tok_v[k, b * CHUNK + r, sl] + p
        return carry2
      lax.fori_loop(0, nvec // UNROLL, vec_body, 0)
      return carry
    lax.fori_loop(0, CHUNK, row_body, 0)

  pend_g = {0: issue_gathers(0, 0)}
  if nch > 1:
    pend_g[1] = issue_gathers(1, 1)
  pend_w = {}
  for c in range(nch):
    k = c % NBUF
    for cp in pend_g.pop(c):
      cp.wait()
    add_pos(k)
    pend_w[c] = issue_writes(k, c)
    nxt = c + 2
    if nxt < nch:
      kk = nxt % NBUF
      prev = nxt - NBUF
      if prev in pend_w:
        for cp in pend_w.pop(prev):
          cp.wait()
      pend_g[nxt] = issue_gathers(kk, nxt)
  for cps in pend_w.values():
    for cp in cps:
      cp.wait()


@functools.partial(jax.jit, static_argnames=("nbatch", "seqlen", "dim"))
def _embed(seq_perm, token_table, pos_table, *, nbatch, seqlen, dim):
  pw = seqlen // NUM_WORKERS
  nch = pw // CHUNK
  mesh = plsc.VectorSubcoreMesh(core_axis_name="c", subcore_axis_name="s")
  kfn = pl.kernel(
      functools.partial(_embed_body, nbatch, seqlen, dim),
      mesh=mesh,
      out_type=jax.ShapeDtypeStruct((nbatch, seqlen, dim), jnp.float32),
      scratch_types=[
          pltpu.VMEM((nch, nbatch * CHUNK), jnp.int32),
          pltpu.VMEM((NBUF, CHUNK, dim), jnp.float32),
          pltpu.VMEM((NBUF, nbatch * CHUNK, dim), jnp.float32),
      ] + [pltpu.SemaphoreType.DMA] * (2 * NBUF),
  )
  return kfn(seq_perm, token_table, pos_table)


def kernel(sequence, token_table, pos_table):
  b, s = sequence.shape
  dim = token_table.shape[1]
  pw = s // NUM_WORKERS
  nch = pw // CHUNK
  seq_perm = jnp.transpose(
      sequence.astype(jnp.int32).reshape(b, NUM_WORKERS, nch, CHUNK),
      (1, 2, 0, 3)).reshape(NUM_WORKERS, nch, b * CHUNK)
  return _embed(seq_perm, token_table, pos_table,
                nbatch=b, seqlen=s, dim=dim)


# speedup vs baseline: 1.4637x; 1.4637x over previous
import functools

import jax
import jax.numpy as jnp
from jax import lax
from jax.experimental import pallas as pl
from jax.experimental.pallas import tpu as pltpu
from jax.experimental.pallas import tpu_sc as plsc

NUM_CORES = 2
NUM_SUBCORES = 16
NUM_WORKERS = NUM_CORES * NUM_SUBCORES
LANES = 16
CHUNK = 8
NBUF = 3


def _embed_body(nbatch, seqlen, dim, seqp_hbm, tok_hbm, pos_hbm, out_hbm,
                idx_v, pos_v, tok_v, *sems):
  gs = sems[:NBUF]
  ws = sems[NBUF:]
  pw = seqlen // NUM_WORKERS
  nch = pw // CHUNK
  nvec = dim // LANES

  wid = lax.axis_index("s") * NUM_CORES + lax.axis_index("c")
  p0 = pl.multiple_of(wid * pw, pw)

  pltpu.sync_copy(seqp_hbm.at[wid], idx_v)

  def issue_gathers(k, c):
    return [
        pltpu.async_copy(pos_hbm.at[pl.ds(p0 + c * CHUNK, CHUNK)],
                         pos_v.at[k], gs[k]),
        pltpu.async_copy(tok_hbm.at[idx_v.at[c]], tok_v.at[k], gs[k]),
    ]

  def issue_writes(k, c):
    return [pltpu.async_copy(tok_v.at[k, pl.ds(b * CHUNK, CHUNK)],
                             out_hbm.at[b, pl.ds(p0 + c * CHUNK, CHUNK)],
                             ws[k])
            for b in range(nbatch)]

  def add_pos(k):
    def row_body(r, carry):
      for v in range(nvec):
        sl = pl.ds(v * LANES, LANES)
        p = pos_v[k, r, sl]
        for b in range(nbatch):
          tok_v[k, b * CHUNK + r, sl] = tok_v[k, b * CHUNK + r, sl] + p
      return carry
    lax.fori_loop(0, CHUNK, row_body, 0)

  pend_g = {0: issue_gathers(0, 0)}
  if nch > 1:
    pend_g[1] = issue_gathers(1, 1)
  pend_w = {}
  for c in range(nch):
    k = c % NBUF
    for cp in pend_g.pop(c):
      cp.wait()
    nxt = c + 2
    if nxt < nch:
      kk = nxt % NBUF
      prev = nxt - NBUF
      if prev in pend_w:
        for cp in pend_w.pop(prev):
          cp.wait()
      pend_g[nxt] = issue_gathers(kk, nxt)
    add_pos(k)
    pend_w[c] = issue_writes(k, c)
  for cps in pend_w.values():
    for cp in cps:
      cp.wait()


@functools.partial(jax.jit, static_argnames=("nbatch", "seqlen", "dim"))
def _embed(seq_perm, token_table, pos_table, *, nbatch, seqlen, dim):
  pw = seqlen // NUM_WORKERS
  nch = pw // CHUNK
  mesh = plsc.VectorSubcoreMesh(core_axis_name="c", subcore_axis_name="s")
  kfn = pl.kernel(
      functools.partial(_embed_body, nbatch, seqlen, dim),
      mesh=mesh,
      out_type=jax.ShapeDtypeStruct((nbatch, seqlen, dim), jnp.float32),
      scratch_types=[
          pltpu.VMEM((nch, nbatch * CHUNK), jnp.int32),
          pltpu.VMEM((NBUF, CHUNK, dim), jnp.float32),
          pltpu.VMEM((NBUF, nbatch * CHUNK, dim), jnp.float32),
      ] + [pltpu.SemaphoreType.DMA] * (2 * NBUF),
  )
  return kfn(seq_perm, token_table, pos_table)


def kernel(sequence, token_table, pos_table):
  b, s = sequence.shape
  dim = token_table.shape[1]
  pw = s // NUM_WORKERS
  nch = pw // CHUNK
  seq_perm = jnp.transpose(
      sequence.astype(jnp.int32).reshape(b, NUM_WORKERS, nch, CHUNK),
      (1, 2, 0, 3)).reshape(NUM_WORKERS, nch, b * CHUNK)
  return _embed(seq_perm, token_table, pos_table,
                nbatch=b, seqlen=s, dim=dim)
